# baseline (speedup 1.0000x reference)
import functools

import jax
import jax.numpy as jnp
from jax.experimental import pallas as pl
from jax.experimental.pallas import tpu as pltpu

_CHUNKS = ((0, 8),)


def _gc_kernel_body(adj_hbm, x_hbm, w_hbm, out_hbm,
                    x_buf, w_buf, adj_buf, o_buf,
                    adj_sem, x_sem, w_sem, out_sem,
                    *, W, npairs):
    i = pl.program_id(0)
    p0 = i * npairs

    def chunk_copy(c):
        s, e = _CHUNKS[c]
        return pltpu.make_async_copy(
            adj_hbm.at[pl.ds(p0 + s, e - s)], adj_buf.at[pl.ds(s, e - s)],
            adj_sem.at[c])

    def out_copy(k):
        return pltpu.make_async_copy(
            o_buf.at[k], out_hbm.at[p0 + k], out_sem.at[k])

    x_copy = pltpu.make_async_copy(x_hbm.at[pl.ds(p0, npairs)], x_buf, x_sem)
    w_copy = pltpu.make_async_copy(w_hbm, w_buf, w_sem)

    chunk_copy(0).start()
    x_copy.start()
    w_copy.start()
    for c in range(1, len(_CHUNKS)):
        chunk_copy(c).start()
    x_copy.wait()
    w_copy.wait()

    chunk_of_pair = []
    for c, (s, e) in enumerate(_CHUNKS):
        chunk_of_pair += [c] * (e - s)

    waited = set()
    for k in range(npairs):
        c = chunk_of_pair[k]
        if c not in waited:
            chunk_copy(c).wait()
            waited.add(c)
        xw = jnp.dot(x_buf[k], w_buf[k % W],
                     preferred_element_type=jnp.float32)
        o_buf[k] = jnp.dot(adj_buf[k], xw,
                           preferred_element_type=jnp.float32)
        out_copy(k).start()

    for k in range(npairs):
        out_copy(k).wait()


def kernel(adjacency, nodes, weights):
    B, W, N, _ = adjacency.shape
    Fin = nodes.shape[-1]
    Fout = weights.shape[-1]
    itemsize = jnp.dtype(adjacency.dtype).itemsize
    npairs = (B * W) // 2

    flops = 2 * B * W * (N * N * Fout + N * Fin * Fout)
    bytes_accessed = itemsize * (adjacency.size + nodes.size + weights.size
                                 + B * W * N * Fout)
    cost = pl.CostEstimate(flops=flops, transcendentals=0,
                           bytes_accessed=bytes_accessed)

    body = functools.partial(_gc_kernel_body, W=W, npairs=npairs)

    out_flat = pl.pallas_call(
        body,
        out_shape=jax.ShapeDtypeStruct((B * W, N, Fout), nodes.dtype),
        grid=(2,),
        in_specs=[
            pl.BlockSpec(memory_space=pl.ANY),
            pl.BlockSpec(memory_space=pl.ANY),
            pl.BlockSpec(memory_space=pl.ANY),
        ],
        out_specs=pl.BlockSpec(memory_space=pl.ANY),
        scratch_shapes=[
            pltpu.VMEM((npairs, N, Fin), jnp.float32),
            pltpu.VMEM((W, Fin, Fout), jnp.float32),
            pltpu.VMEM((npairs, N, N), jnp.float32),
            pltpu.VMEM((npairs, N, Fout), jnp.float32),
            pltpu.SemaphoreType.DMA((len(_CHUNKS),)),
            pltpu.SemaphoreType.DMA,
            pltpu.SemaphoreType.DMA,
            pltpu.SemaphoreType.DMA((npairs,)),
        ],
        compiler_params=pltpu.CompilerParams(
            dimension_semantics=("parallel",),
            vmem_limit_bytes=48 * 1024 * 1024,
        ),
        cost_estimate=cost,
    )(adjacency.reshape(B * W, N, N), nodes.reshape(B * W, N, Fin), weights)

    return out_flat.reshape(B, W, N, Fout)


# speedup vs baseline: 1.2224x; 1.2224x over previous
import jax
import jax.numpy as jnp
from jax.experimental import pallas as pl
from jax.experimental.pallas import tpu as pltpu


def _gc_kernel(adj_a_ref, adj_b_ref, x_ref, w_ref, out_ref):
    W = w_ref.shape[0]
    step = pl.program_id(0)
    for k in range(4):
        adj = adj_a_ref[k] if k < 2 else adj_b_ref[k - 2]
        xw = jnp.dot(x_ref[k], w_ref[k % W],
                     preferred_element_type=jnp.float32)
        y = jnp.dot(adj, xw, preferred_element_type=jnp.float32)
        out_ref[k] = y.astype(out_ref.dtype)


def kernel(adjacency, nodes, weights):
    B, W, N, _ = adjacency.shape
    Fin = nodes.shape[-1]
    Fout = weights.shape[-1]
    itemsize = jnp.dtype(adjacency.dtype).itemsize
    P = B * W

    flops = 2 * B * W * (N * N * Fout + N * Fin * Fout)
    bytes_accessed = itemsize * (adjacency.size + nodes.size + weights.size
                                 + B * W * N * Fout)
    cost = pl.CostEstimate(flops=flops, transcendentals=0,
                           bytes_accessed=bytes_accessed)

    adj_flat = adjacency.reshape(P, N, N)
    x_flat = nodes.reshape(P, N, Fin)

    out_flat = pl.pallas_call(
        _gc_kernel,
        out_shape=jax.ShapeDtypeStruct((P, N, Fout), nodes.dtype),
        grid_spec=pl.GridSpec(
            grid=(P // 4,),
            in_specs=[
                pl.BlockSpec((2, N, N), lambda s: (2 * s, 0, 0)),
                pl.BlockSpec((2, N, N), lambda s: (2 * s + 1, 0, 0)),
                pl.BlockSpec((4, N, Fin), lambda s: (s, 0, 0)),
                pl.BlockSpec((W, Fin, Fout), lambda s: (0, 0, 0)),
            ],
            out_specs=pl.BlockSpec((4, N, Fout), lambda s: (s, 0, 0)),
        ),
        compiler_params=pltpu.CompilerParams(
            dimension_semantics=("arbitrary",),
            vmem_limit_bytes=48 * 1024 * 1024,
        ),
        cost_estimate=cost,
    )(adj_flat, adj_flat, x_flat, weights)

    return out_flat.reshape(B, W, N, Fout)


# speedup vs baseline: 1.3963x; 1.1423x over previous
import functools

import jax
import jax.numpy as jnp
from jax.experimental import pallas as pl
from jax.experimental.pallas import tpu as pltpu

_RCHUNK = 2
_WCHUNK = 4


def _gc_kernel_body(adj_hbm, x_hbm, w_hbm, out_hbm,
                    x_buf, w_buf, adj_buf, o_buf,
                    adj_sem, x_sem, w_sem, out_sem, *, W, P):
    n_rchunks = P // _RCHUNK
    n_wchunks = P // _WCHUNK

    def adj_copy(c):
        sl = pl.ds(c * _RCHUNK, _RCHUNK)
        return pltpu.make_async_copy(adj_hbm.at[sl], adj_buf.at[sl],
                                     adj_sem.at[c])

    def out_copy(c):
        sl = pl.ds(c * _WCHUNK, _WCHUNK)
        return pltpu.make_async_copy(o_buf.at[sl], out_hbm.at[sl],
                                     out_sem.at[c])

    x_copy = pltpu.make_async_copy(x_hbm, x_buf, x_sem)
    w_copy = pltpu.make_async_copy(w_hbm, w_buf, w_sem)

    x_copy.start()
    w_copy.start()
    for c in range(n_rchunks):
        adj_copy(c).start()
    x_copy.wait()
    w_copy.wait()

    for k in range(P):
        if k % _RCHUNK == 0:
            adj_copy(k // _RCHUNK).wait()
        xw = jnp.dot(x_buf[k], w_buf[k % W],
                     preferred_element_type=jnp.float32)
        o_buf[k] = jnp.dot(adj_buf[k], xw,
                           preferred_element_type=jnp.float32)
        if (k + 1) % _WCHUNK == 0:
            out_copy(k // _WCHUNK).start()

    for c in range(n_wchunks):
        out_copy(c).wait()


def kernel(adjacency, nodes, weights):
    B, W, N, _ = adjacency.shape
    Fin = nodes.shape[-1]
    Fout = weights.shape[-1]
    itemsize = jnp.dtype(adjacency.dtype).itemsize
    P = B * W

    flops = 2 * B * W * (N * N * Fout + N * Fin * Fout)
    bytes_accessed = itemsize * (adjacency.size + nodes.size + weights.size
                                 + B * W * N * Fout)
    cost = pl.CostEstimate(flops=flops, transcendentals=0,
                           bytes_accessed=bytes_accessed)

    body = functools.partial(_gc_kernel_body, W=W, P=P)

    out_flat = pl.pallas_call(
        body,
        out_shape=jax.ShapeDtypeStruct((P, N, Fout), nodes.dtype),
        grid=(1,),
        in_specs=[
            pl.BlockSpec(memory_space=pl.ANY),
            pl.BlockSpec(memory_space=pl.ANY),
            pl.BlockSpec(memory_space=pl.ANY),
        ],
        out_specs=pl.BlockSpec(memory_space=pl.ANY),
        scratch_shapes=[
            pltpu.VMEM((P, N, Fin), jnp.float32),
            pltpu.VMEM((W, Fin, Fout), jnp.float32),
            pltpu.VMEM((P, N, N), jnp.float32),
            pltpu.VMEM((P, N, Fout), jnp.float32),
            pltpu.SemaphoreType.DMA((P // _RCHUNK,)),
            pltpu.SemaphoreType.DMA,
            pltpu.SemaphoreType.DMA,
            pltpu.SemaphoreType.DMA((P // _WCHUNK,)),
        ],
        compiler_params=pltpu.CompilerParams(
            dimension_semantics=("arbitrary",),
            vmem_limit_bytes=52 * 1024 * 1024,
        ),
        cost_estimate=cost,
    )(adjacency.reshape(P, N, N), nodes.reshape(P, N, Fin), weights)

    return out_flat.reshape(B, W, N, Fout)
